# baseline (speedup 1.0000x reference)
import jax
import jax.numpy as jnp
from jax import lax
from jax.experimental import pallas as pl
from jax.experimental.pallas import tpu as pltpu
from jax.experimental.pallas import tpu_sc as plsc

VOCAB = 1000000
DIM = 64
ROWS = 16384
COLS = 50

_INFO = plsc.get_sparse_core_info()
NC = _INFO.num_cores
NS = _INFO.num_subcores
NW = NC * NS

NGRP = VOCAB // 128
NPAIR = NGRP // 2
T2_ROWS = VOCAB // 2
TAIL_T2 = NGRP * 64

BLK = 128
NBLK = ROWS * COLS // BLK
BLK_PER_W = NBLK // NW
RB = ROWS // BLK


def _iota16():
    return lax.iota(jnp.int32, 16)


def _transpose_pairs(buf, outb):
    rows = [_iota16() + 16 * q for q in range(4)]

    @pl.loop(0, 64, unroll=4)
    def _(m):
        ca = jnp.full((16,), 2 * m, jnp.int32)
        cb = ca + 1
        for q in range(4):
            outb[m, pl.ds(16 * q, 16)] = plsc.load_gather(buf, [rows[q], ca])
        for q in range(4):
            outb[m, pl.ds(64 + 16 * q, 16)] = plsc.load_gather(buf, [rows[q], cb])


def _stage1_body(tT_hbm, tail_hbm, t2_hbm, buf_v, out_v, isems, osems):
    wid = lax.axis_index("s") * NC + lax.axis_index("c")
    p0 = wid * (NPAIR // NW) + jnp.minimum(wid, NPAIR % NW)
    npair = (NPAIR // NW) + jnp.where(wid < NPAIR % NW, 1, 0)
    g0 = 2 * p0

    def in_fire(g, b):
        pltpu.async_copy(tT_hbm.at[:, pl.ds(g * 128, 128)], buf_v.at[b],
                         isems.at[b])

    def in_wait(b):
        pltpu.make_async_copy(tT_hbm.at[:, pl.ds(0, 128)], buf_v.at[b],
                              isems.at[b]).wait()

    def out_fire(g, b):
        pltpu.async_copy(out_v.at[b], t2_hbm.at[pl.ds(g * 64, 64)],
                         osems.at[b])

    def out_wait(b):
        pltpu.make_async_copy(out_v.at[0], t2_hbm.at[pl.ds(0, 64)],
                              osems.at[b]).wait()

    in_fire(g0, 0)

    @pl.loop(0, npair)
    def _(pk):
        g = g0 + 2 * pk
        for b in range(2):
            @pl.when(jnp.logical_or(b == 0, pk < npair - 1))
            def _():
                in_fire(g + b + 1, 1 - b)
            in_wait(b)

            @pl.when(pk > 0)
            def _():
                out_wait(b)
            _transpose_pairs(buf_v.at[b], out_v.at[b])
            out_fire(g + b, b)

    out_wait(0)
    out_wait(1)

    @pl.when(wid == 0)
    def _():
        pltpu.sync_copy(tail_hbm, buf_v.at[0, pl.ds(0, 32), :])
        pltpu.sync_copy(buf_v.at[0, pl.ds(0, 32), :],
                        t2_hbm.at[pl.ds(TAIL_T2, 32)])


def _prep_block(idxr, idx2, colb):
    for s in range(8):
        v = idxr[pl.ds(16 * s, 16)]
        idx2[pl.ds(16 * s, 16)] = lax.shift_right_logical(v, 1)
        colb[pl.ds(16 * s, 16)] = lax.shift_left(jnp.bitwise_and(v, 1), 6)


def _transpose_select(g_v, colb, outb):
    rows = [_iota16() + 16 * q for q in range(8)]
    for q in range(8):
        cb = colb[pl.ds(16 * q, 16)]

        @pl.loop(0, 64, unroll=4)
        def _(d):
            outb[d, pl.ds(16 * q, 16)] = plsc.load_gather(
                g_v, [rows[q], cb + d])


def _stage2_body(t2_hbm, xT_hbm, out_hbm, idxr_v, idx2_v, colb_v, g_v,
                 out_v, xsems, gsems, osems):
    wid = lax.axis_index("s") * NC + lax.axis_index("c")
    t0 = wid * BLK_PER_W

    def x_fire(t, b):
        c = t // RB
        r0 = (t % RB) * BLK
        pltpu.async_copy(xT_hbm.at[c, pl.ds(r0, BLK)], idxr_v.at[b],
                         xsems.at[b])

    def x_wait(b):
        pltpu.make_async_copy(xT_hbm.at[0, pl.ds(0, BLK)], idxr_v.at[b],
                              xsems.at[b]).wait()

    def g_fire(b):
        pltpu.async_copy(t2_hbm.at[idx2_v.at[b]], g_v.at[b], gsems.at[b])

    def g_wait(b):
        pltpu.make_async_copy(t2_hbm.at[idx2_v.at[b]], g_v.at[b],
                              gsems.at[b]).wait()

    def o_fire(t, b):
        c = t // RB
        r0 = (t % RB) * BLK
        pltpu.async_copy(out_v.at[b], out_hbm.at[c, :, pl.ds(r0, BLK)],
                         osems.at[b])

    def o_wait(b):
        pltpu.make_async_copy(out_v.at[0], out_hbm.at[0, :, pl.ds(0, BLK)],
                              osems.at[b]).wait()

    x_fire(t0, 0)
    x_wait(0)
    _prep_block(idxr_v.at[0], idx2_v.at[0], colb_v.at[0])
    g_fire(0)
    x_fire(t0 + 1, 1)

    def step(t, b, prefetch, first_pair):
        x_wait(1 - b)
        _prep_block(idxr_v.at[1 - b], idx2_v.at[1 - b], colb_v.at[1 - b])
        g_fire(1 - b)
        if prefetch:
            x_fire(t + 2, b)
        g_wait(b)
        if not first_pair:
            o_wait(b)
        _transpose_select(g_v.at[b], colb_v.at[b], out_v.at[b])
        o_fire(t, b)

    step(t0, 0, True, True)
    step(t0 + 1, 1, True, True)

    @pl.loop(1, BLK_PER_W // 2 - 1)
    def _(pk):
        t = t0 + 2 * pk
        step(t, 0, True, False)
        step(t + 1, 1, True, False)

    t_last = t0 + BLK_PER_W - 2
    step(t_last, 0, False, False)
    g_wait(1)
    o_wait(1)
    _transpose_select(g_v.at[1], colb_v.at[1], out_v.at[1])
    o_fire(t_last + 1, 1)
    o_wait(0)
    o_wait(1)


_MESH = dict(core_axis_name="c", subcore_axis_name="s")


@jax.jit
def _sc_embed(tT, tail, xT):
    s1 = pl.kernel(
        _stage1_body,
        out_type=jax.ShapeDtypeStruct((T2_ROWS, 128), jnp.float32),
        mesh=plsc.VectorSubcoreMesh(**_MESH),
        scratch_types=[
            pltpu.VMEM((2, 64, 128), jnp.float32),
            pltpu.VMEM((2, 64, 128), jnp.float32),
            pltpu.SemaphoreType.DMA((2,)),
            pltpu.SemaphoreType.DMA((2,)),
        ],
        compiler_params=pltpu.CompilerParams(use_tc_tiling_on_sc=True,
                                             needs_layout_passes=False),
    )
    t2 = s1(tT, tail)
    s2 = pl.kernel(
        _stage2_body,
        out_type=jax.ShapeDtypeStruct((COLS, DIM, ROWS), jnp.float32),
        mesh=plsc.VectorSubcoreMesh(**_MESH),
        scratch_types=[
            pltpu.VMEM((2, BLK), jnp.int32),
            pltpu.VMEM((2, BLK), jnp.int32),
            pltpu.VMEM((2, BLK), jnp.int32),
            pltpu.VMEM((2, BLK, 128), jnp.float32),
            pltpu.VMEM((2, DIM, BLK), jnp.float32),
            pltpu.SemaphoreType.DMA((2,)),
            pltpu.SemaphoreType.DMA((2,)),
            pltpu.SemaphoreType.DMA((2,)),
        ],
        compiler_params=pltpu.CompilerParams(use_tc_tiling_on_sc=True,
                                             needs_layout_passes=False),
    )
    return s2(t2, xT)


def kernel(x, table):
    tT = table.T
    tail = table[NGRP * 128:].reshape(32, 128)
    xT = x.astype(jnp.int32).T
    out_phys = _sc_embed(tT, tail, xT)
    return out_phys.transpose(2, 0, 1)
